# Initial kernel scaffold; baseline (speedup 1.0000x reference)
import jax
import jax.numpy as jnp
from jax.experimental import pallas as pl


def kernel(points, params):
    raise NotImplementedError("write your pallas kernel here")


# speedup vs baseline: 3.8760x; 3.8760x over previous
import functools

import numpy as np
import jax
import jax.numpy as jnp
from jax import lax
from jax.experimental import pallas as pl
from jax.experimental.pallas import tpu as pltpu
from jax.experimental.pallas import tpu_sc as plsc

_B = 2
_N = 8192
_RADII = [[0.2, 0.8], [0.8, 1.6], [1.6, 4.8], [4.8, 6.4]]
_NSAMPLE = [16, 32]
_MAXT = (3.0, 3.0, 2.0)


def _fps(xyz_b, m):
    bsz, n, _ = xyz_b.shape
    nr = n // 128
    planes = xyz_b.transpose(0, 2, 1).reshape(bsz, 3, nr, 128)

    def body(pl_ref, rows_ref, o_ref):
        x = pl_ref[0, 0]
        y = pl_ref[0, 1]
        z = pl_ref[0, 2]
        flat = (lax.broadcasted_iota(jnp.int32, (nr, 128), 0) * 128
                + lax.broadcasted_iota(jnp.int32, (nr, 128), 1))

        def step(i, carry):
            dist, far = carry
            row = rows_ref[0, pl.ds(far, 1), :]
            o_ref[0, pl.ds(i, 1), :] = row
            d = (x - row[0, 0]) ** 2 + (y - row[0, 1]) ** 2 + (z - row[0, 2]) ** 2
            dist = jnp.minimum(dist, d)
            mx = jnp.max(dist)
            far2 = jnp.min(jnp.where(dist == mx, flat, jnp.int32(n)))
            return dist, far2

        lax.fori_loop(0, m, step,
                      (jnp.full((nr, 128), 1e10, jnp.float32), jnp.int32(0)))

    return pl.pallas_call(
        body,
        grid=(bsz,),
        in_specs=[pl.BlockSpec((1, 3, nr, 128), lambda b: (b, 0, 0, 0)),
                  pl.BlockSpec((1, n, 3), lambda b: (b, 0, 0))],
        out_specs=pl.BlockSpec((1, m, 3), lambda b: (b, 0, 0)),
        out_shape=jax.ShapeDtypeStruct((bsz, m, 3), jnp.float32),
    )(planes, xyz_b)


def _knn(q, keys, k=32):
    bsz, s, _ = q.shape
    n = keys.shape[1]
    kp = keys.transpose(0, 2, 1)
    sb = 8

    def body(q_ref, k_ref, gi_ref, dv_ref, d_sc):
        qv = q_ref[0]
        kx = k_ref[0, 0:1, :]
        ky = k_ref[0, 1:2, :]
        kz = k_ref[0, 2:3, :]
        d_sc[...] = ((qv[:, 0:1] - kx) ** 2 + (qv[:, 1:2] - ky) ** 2
                     + (qv[:, 2:3] - kz) ** 2)
        lane_k = lax.broadcasted_iota(jnp.int32, (sb, k), 1)
        flat = lax.broadcasted_iota(jnp.int32, (sb, n), 1)

        def step(t, carry):
            gi, dv = carry
            d = d_sc[...]
            m = jnp.min(d, axis=1, keepdims=True)
            sel = jnp.min(jnp.where(d == m, flat, jnp.int32(n)),
                          axis=1, keepdims=True)
            gi = jnp.where(lane_k == t, sel, gi)
            dv = jnp.where(lane_k == t, m, dv)
            d_sc[...] = jnp.where(flat == sel, jnp.float32(np.inf), d)
            return gi, dv

        gi, dv = lax.fori_loop(
            0, k, step,
            (jnp.zeros((sb, k), jnp.int32), jnp.zeros((sb, k), jnp.float32)))
        gi_ref[0] = gi
        dv_ref[0] = dv

    return pl.pallas_call(
        body,
        grid=(bsz, s // sb),
        in_specs=[pl.BlockSpec((1, sb, 3), lambda b, i: (b, i, 0)),
                  pl.BlockSpec((1, 3, n), lambda b, i: (b, 0, 0))],
        out_specs=[pl.BlockSpec((1, sb, k), lambda b, i: (b, i, 0)),
                   pl.BlockSpec((1, sb, k), lambda b, i: (b, i, 0))],
        out_shape=[jax.ShapeDtypeStruct((bsz, s, k), jnp.int32),
                   jax.ShapeDtypeStruct((bsz, s, k), jnp.float32)],
        scratch_shapes=[pltpu.VMEM((sb, n), jnp.float32)],
    )(q, kp)


def _gather_rows(table, idx):
    v, d = table.shape
    m = idx.shape[0]
    nw = 32
    b_per_w = m // nw
    rpc = 128
    nch = b_per_w // rpc
    mesh = plsc.VectorSubcoreMesh(core_axis_name="c", subcore_axis_name="s")

    @functools.partial(
        pl.kernel, mesh=mesh,
        out_type=jax.ShapeDtypeStruct((m, d), jnp.float32),
        scratch_types=[pltpu.VMEM((rpc,), jnp.int32),
                       pltpu.VMEM((rpc, d), jnp.float32),
                       pltpu.SemaphoreType.DMA],
    )
    def k(table_hbm, idx_hbm, out_hbm, idx_v, rows_v, sem):
        wid = lax.axis_index("s") * 2 + lax.axis_index("c")
        base = wid * b_per_w

        def chunk(ci, carry):
            off = base + ci * rpc
            pltpu.sync_copy(idx_hbm.at[pl.ds(off, rpc)], idx_v)
            pltpu.async_copy(table_hbm.at[idx_v], rows_v, sem).wait()
            pltpu.sync_copy(rows_v, out_hbm.at[pl.ds(off, rpc)])
            return carry

        lax.fori_loop(0, nch, chunk, 0)

    return k(table, idx)


def _gather(xyz, feats, gi, dpad):
    bsz, n, _ = xyz.shape
    _, s, kk = gi.shape
    tab = jnp.concatenate([xyz, feats], axis=-1)
    c = tab.shape[-1]
    tab = jnp.pad(tab, ((0, 0), (0, 0), (0, dpad - c)))
    tab2 = tab.reshape(bsz * n, dpad)
    offs = (jnp.arange(bsz, dtype=jnp.int32) * n)[:, None, None]
    flat_idx = (gi + offs).reshape(bsz * s * kk)
    out = _gather_rows(tab2, flat_idx)
    return out.reshape(bsz, s, kk, dpad)


def _sa_mlp(gath, ctr, dv, p, cin, radii):
    bsz, s, kk, dp = gath.shape
    sb = 64
    ws, shapes = [], []
    for layers in p['scales']:
        for (w, b) in layers:
            ws.append(w)
            ws.append(b.reshape(1, -1))
    wa, ba = p['agg']
    ws.append(wa)
    ws.append(ba.reshape(1, -1))
    cout = wa.shape[1]
    nlay = [len(layers) for layers in p['scales']]

    def body(g_ref, c_ref, dv_ref, *refs):
        o_ref = refs[-1]
        wr = refs[:-1]
        g = g_ref[0]
        cc = c_ref[0]
        dvv = dv_ref[0]
        rel = g[:, :, 0:3] - cc[:, None, :]
        feat = g[:, :, 3:3 + cin]
        inp = jnp.concatenate([rel, feat], axis=-1)
        outs = []
        wi = 0
        for si, r in enumerate(radii):
            ns = _NSAMPLE[si]
            x = inp[:, :ns, :].reshape(sb * ns, cin + 3)
            for _ in range(nlay[si]):
                w = wr[wi][...]
                b = wr[wi + 1][...]
                wi += 2
                x = jnp.maximum(
                    jnp.dot(x, w, preferred_element_type=jnp.float32) + b, 0.0)
            ck = x.shape[1]
            x3 = x.reshape(sb, ns, ck)
            r2 = np.float32(r * r)
            valid = dvv[:, :ns] <= r2
            acc = x3[:, 0, :]
            for t in range(1, ns):
                vt = valid[:, t:t + 1]
                acc = jnp.maximum(acc, jnp.where(vt, x3[:, t, :], 0.0))
            outs.append(acc)
        cat = jnp.concatenate(outs, axis=-1)
        w = wr[wi][...]
        b = wr[wi + 1][...]
        o_ref[0] = jnp.maximum(
            jnp.dot(cat, w, preferred_element_type=jnp.float32) + b, 0.0)

    w_specs = [pl.BlockSpec(w.shape, lambda b, i: (0,) * w.ndim) for w in ws]
    return pl.pallas_call(
        body,
        grid=(bsz, s // sb),
        in_specs=[pl.BlockSpec((1, sb, kk, dp), lambda b, i: (b, i, 0, 0)),
                  pl.BlockSpec((1, sb, 3), lambda b, i: (b, i, 0)),
                  pl.BlockSpec((1, sb, kk), lambda b, i: (b, i, 0))] + w_specs,
        out_specs=pl.BlockSpec((1, sb, cout), lambda b, i: (b, i, 0)),
        out_shape=jax.ShapeDtypeStruct((bsz, s, cout), jnp.float32),
    )(gath, ctr, dv, *ws)


def _vote(xyz, feats, p):
    bsz, s, c = feats.shape
    w1, b1 = p['mlp']
    wr, br = p['reg']
    wr3 = wr[:, :3]
    br3 = br[:3].reshape(1, 3)
    b1r = b1.reshape(1, -1)

    def body(x_ref, f_ref, w1_ref, b1_ref, wr_ref, br_ref, c_ref, o_ref):
        f = f_ref[0]
        f1 = jnp.maximum(
            jnp.dot(f, w1_ref[...], preferred_element_type=jnp.float32)
            + b1_ref[...], 0.0)
        off = jnp.dot(f1, wr_ref[...], preferred_element_type=jnp.float32) \
            + br_ref[...]
        lane3 = lax.broadcasted_iota(jnp.int32, (1, 3), 1)
        mt = jnp.where(lane3 == 2, jnp.float32(2.0), jnp.float32(3.0))
        lim = jnp.clip(off, -mt, mt)
        c_ref[0] = x_ref[0] + lim
        o_ref[0] = off

    return pl.pallas_call(
        body,
        grid=(bsz,),
        in_specs=[pl.BlockSpec((1, s, 3), lambda b: (b, 0, 0)),
                  pl.BlockSpec((1, s, c), lambda b: (b, 0, 0)),
                  pl.BlockSpec(w1.shape, lambda b: (0, 0)),
                  pl.BlockSpec(b1r.shape, lambda b: (0, 0)),
                  pl.BlockSpec(wr3.shape, lambda b: (0, 0)),
                  pl.BlockSpec(br3.shape, lambda b: (0, 0))],
        out_specs=[pl.BlockSpec((1, s, 3), lambda b: (b, 0, 0)),
                   pl.BlockSpec((1, s, 3), lambda b: (b, 0, 0))],
        out_shape=[jax.ShapeDtypeStruct((bsz, s, 3), jnp.float32),
                   jax.ShapeDtypeStruct((bsz, s, 3), jnp.float32)],
    )(xyz, feats, w1, b1r, wr3, br3)


def _sa_layer(xyz, feats, npoint, li, p, ctr_xyz, dpad):
    cin = feats.shape[-1]
    if ctr_xyz is None:
        new_xyz = _fps(xyz, npoint)
    else:
        new_xyz = ctr_xyz
    gi, dv = _knn(new_xyz, xyz)
    gath = _gather(xyz, feats, gi, dpad)
    nf = _sa_mlp(gath, new_xyz, dv, p, cin, _RADII[li])
    return new_xyz, nf


def kernel(points, params):
    xyz0 = points[:, 1:4].reshape(_B, _N, 3)
    feat0 = points[:, 4:].reshape(_B, _N, -1)
    sa = params['sa']

    nx1, f1 = _sa_layer(xyz0, feat0, 2048, 0, sa[0], None, 128)
    nx2, f2 = _sa_layer(nx1, f1, 512, 1, sa[1], None, 128)
    nx3, f3 = _sa_layer(nx2, f2, 256, 2, sa[2], None, 256)
    centers, offs = _vote(nx3, f3, params['vote'])
    _, cf = _sa_layer(nx3, f3, None, 3, sa[3], centers, 384)

    s = centers.shape[1]
    bidx = jnp.repeat(jnp.arange(_B, dtype=jnp.float32), s)[:, None]
    centers_cat = jnp.concatenate([bidx, centers.reshape(-1, 3)], axis=1)
    origin_cat = jnp.concatenate([bidx, nx3.reshape(-1, 3)], axis=1)
    off_cat = jnp.concatenate([bidx, offs.reshape(-1, 3)], axis=1)
    cf2 = cf.reshape(_B * s, -1)
    return centers_cat, origin_cat, off_cat, cf2


# speedup vs baseline: 6.0110x; 1.5508x over previous
import functools

import numpy as np
import jax
import jax.numpy as jnp
from jax import lax
from jax.experimental import pallas as pl
from jax.experimental.pallas import tpu as pltpu
from jax.experimental.pallas import tpu_sc as plsc

_B = 2
_N = 8192
_RADII = [[0.2, 0.8], [0.8, 1.6], [1.6, 4.8], [4.8, 6.4]]
_NSAMPLE = [16, 32]
_MAXT = (3.0, 3.0, 2.0)


def _fps(xyz_b, m):
    bsz, n, _ = xyz_b.shape
    nr = n // 128
    planes = xyz_b.transpose(0, 2, 1).reshape(bsz, 3, nr, 128)

    def body(pl_ref, rows_ref, o_ref):
        x = pl_ref[0, 0]
        y = pl_ref[0, 1]
        z = pl_ref[0, 2]
        flat = (lax.broadcasted_iota(jnp.int32, (nr, 128), 0) * 128
                + lax.broadcasted_iota(jnp.int32, (nr, 128), 1))

        def step(i, carry):
            dist, far = carry
            row = rows_ref[0, pl.ds(far, 1), :]
            o_ref[0, pl.ds(i, 1), :] = row
            d = (x - row[0, 0]) ** 2 + (y - row[0, 1]) ** 2 + (z - row[0, 2]) ** 2
            dist = jnp.minimum(dist, d)
            mx = jnp.max(dist)
            far2 = jnp.min(jnp.where(dist == mx, flat, jnp.int32(n)))
            return dist, far2

        lax.fori_loop(0, m, step,
                      (jnp.full((nr, 128), 1e10, jnp.float32), jnp.int32(0)))

    return pl.pallas_call(
        body,
        grid=(bsz,),
        in_specs=[pl.BlockSpec((1, 3, nr, 128), lambda b: (b, 0, 0, 0)),
                  pl.BlockSpec((1, n, 3), lambda b: (b, 0, 0))],
        out_specs=pl.BlockSpec((1, m, 3), lambda b: (b, 0, 0)),
        out_shape=jax.ShapeDtypeStruct((bsz, m, 3), jnp.float32),
    )(planes, xyz_b)


def _knn(q, keys, r2max, k=32):
    bsz, s, _ = q.shape
    n = keys.shape[1]
    kp = keys.transpose(0, 2, 1)
    sb = 8

    def body(q_ref, k_ref, gi_ref, dv_ref, d_sc):
        qv = q_ref[0]
        kx = k_ref[0, 0:1, :]
        ky = k_ref[0, 1:2, :]
        kz = k_ref[0, 2:3, :]
        d_sc[...] = ((qv[:, 0:1] - kx) ** 2 + (qv[:, 1:2] - ky) ** 2
                     + (qv[:, 2:3] - kz) ** 2)
        lane_k = lax.broadcasted_iota(jnp.int32, (sb, k), 1)
        flat = lax.broadcasted_iota(jnp.int32, (sb, n), 1)

        def cond(carry):
            t, _, _, mprev = carry
            return (t < k) & ((t == 0) | (jnp.min(mprev) <= r2max))

        def step(carry):
            t, gi, dv, _ = carry
            d = d_sc[...]
            m = jnp.min(d, axis=1, keepdims=True)
            sel = jnp.min(jnp.where(d == m, flat, jnp.int32(n)),
                          axis=1, keepdims=True)
            gi = jnp.where(lane_k == t, sel, gi)
            dv = jnp.where(lane_k == t, m, dv)
            d_sc[...] = jnp.where(flat == sel, jnp.float32(np.inf), d)
            return t + 1, gi, dv, m

        _, gi, dv, _ = lax.while_loop(
            cond, step,
            (jnp.int32(0), jnp.zeros((sb, k), jnp.int32),
             jnp.full((sb, k), np.inf, jnp.float32),
             jnp.zeros((sb, 1), jnp.float32)))
        gi_ref[0] = gi
        dv_ref[0] = dv

    return pl.pallas_call(
        body,
        grid=(bsz, s // sb),
        in_specs=[pl.BlockSpec((1, sb, 3), lambda b, i: (b, i, 0)),
                  pl.BlockSpec((1, 3, n), lambda b, i: (b, 0, 0))],
        out_specs=[pl.BlockSpec((1, sb, k), lambda b, i: (b, i, 0)),
                   pl.BlockSpec((1, sb, k), lambda b, i: (b, i, 0))],
        out_shape=[jax.ShapeDtypeStruct((bsz, s, k), jnp.int32),
                   jax.ShapeDtypeStruct((bsz, s, k), jnp.float32)],
        scratch_shapes=[pltpu.VMEM((sb, n), jnp.float32)],
    )(q, kp)


def _gather_rows(table, idx):
    v, d = table.shape
    m = idx.shape[0]
    nw = 32
    b_per_w = m // nw
    rpc = 128
    nch = b_per_w // rpc
    mesh = plsc.VectorSubcoreMesh(core_axis_name="c", subcore_axis_name="s")

    @functools.partial(
        pl.kernel, mesh=mesh,
        out_type=jax.ShapeDtypeStruct((m, d), jnp.float32),
        scratch_types=[pltpu.VMEM((rpc,), jnp.int32),
                       pltpu.VMEM((rpc, d), jnp.float32),
                       pltpu.SemaphoreType.DMA],
    )
    def k(table_hbm, idx_hbm, out_hbm, idx_v, rows_v, sem):
        wid = lax.axis_index("s") * 2 + lax.axis_index("c")
        base = wid * b_per_w

        def chunk(ci, carry):
            off = base + ci * rpc
            pltpu.sync_copy(idx_hbm.at[pl.ds(off, rpc)], idx_v)
            pltpu.async_copy(table_hbm.at[idx_v], rows_v, sem).wait()
            pltpu.sync_copy(rows_v, out_hbm.at[pl.ds(off, rpc)])
            return carry

        lax.fori_loop(0, nch, chunk, 0)

    return k(table, idx)


def _gather(xyz, feats, gi, dpad):
    bsz, n, _ = xyz.shape
    _, s, kk = gi.shape
    tab = jnp.concatenate([xyz, feats], axis=-1)
    c = tab.shape[-1]
    tab = jnp.pad(tab, ((0, 0), (0, 0), (0, dpad - c)))
    tab2 = tab.reshape(bsz * n, dpad)
    offs = (jnp.arange(bsz, dtype=jnp.int32) * n)[:, None, None]
    flat_idx = (gi + offs).reshape(bsz * s * kk)
    out = _gather_rows(tab2, flat_idx)
    return out.reshape(bsz, s, kk, dpad)


def _sa_mlp(gath, ctr, dv, p, cin, radii):
    bsz, s, kk, dp = gath.shape
    sb = 64
    ws, shapes = [], []
    for layers in p['scales']:
        for (w, b) in layers:
            ws.append(w)
            ws.append(b.reshape(1, -1))
    wa, ba = p['agg']
    ws.append(wa)
    ws.append(ba.reshape(1, -1))
    cout = wa.shape[1]
    nlay = [len(layers) for layers in p['scales']]

    def body(g_ref, c_ref, dv_ref, *refs):
        o_ref = refs[-1]
        wr = refs[:-1]
        g = g_ref[0]
        cc = c_ref[0]
        dvv = dv_ref[0]
        rel = g[:, :, 0:3] - cc[:, None, :]
        feat = g[:, :, 3:3 + cin]
        inp = jnp.concatenate([rel, feat], axis=-1)
        outs = []
        wi = 0
        for si, r in enumerate(radii):
            ns = _NSAMPLE[si]
            x = inp[:, :ns, :].reshape(sb * ns, cin + 3)
            for _ in range(nlay[si]):
                w = wr[wi][...]
                b = wr[wi + 1][...]
                wi += 2
                x = jnp.maximum(
                    jnp.dot(x, w, preferred_element_type=jnp.float32) + b, 0.0)
            ck = x.shape[1]
            x3 = x.reshape(sb, ns, ck)
            r2 = np.float32(r * r)
            valid = dvv[:, :ns] <= r2
            acc = x3[:, 0, :]
            for t in range(1, ns):
                vt = valid[:, t:t + 1]
                acc = jnp.maximum(acc, jnp.where(vt, x3[:, t, :], 0.0))
            outs.append(acc)
        cat = jnp.concatenate(outs, axis=-1)
        w = wr[wi][...]
        b = wr[wi + 1][...]
        o_ref[0] = jnp.maximum(
            jnp.dot(cat, w, preferred_element_type=jnp.float32) + b, 0.0)

    w_specs = [pl.BlockSpec(w.shape, lambda b, i: (0,) * w.ndim) for w in ws]
    return pl.pallas_call(
        body,
        grid=(bsz, s // sb),
        in_specs=[pl.BlockSpec((1, sb, kk, dp), lambda b, i: (b, i, 0, 0)),
                  pl.BlockSpec((1, sb, 3), lambda b, i: (b, i, 0)),
                  pl.BlockSpec((1, sb, kk), lambda b, i: (b, i, 0))] + w_specs,
        out_specs=pl.BlockSpec((1, sb, cout), lambda b, i: (b, i, 0)),
        out_shape=jax.ShapeDtypeStruct((bsz, s, cout), jnp.float32),
    )(gath, ctr, dv, *ws)


def _vote(xyz, feats, p):
    bsz, s, c = feats.shape
    w1, b1 = p['mlp']
    wr, br = p['reg']
    wr3 = wr[:, :3]
    br3 = br[:3].reshape(1, 3)
    b1r = b1.reshape(1, -1)

    def body(x_ref, f_ref, w1_ref, b1_ref, wr_ref, br_ref, c_ref, o_ref):
        f = f_ref[0]
        f1 = jnp.maximum(
            jnp.dot(f, w1_ref[...], preferred_element_type=jnp.float32)
            + b1_ref[...], 0.0)
        off = jnp.dot(f1, wr_ref[...], preferred_element_type=jnp.float32) \
            + br_ref[...]
        lane3 = lax.broadcasted_iota(jnp.int32, (1, 3), 1)
        mt = jnp.where(lane3 == 2, jnp.float32(2.0), jnp.float32(3.0))
        lim = jnp.clip(off, -mt, mt)
        c_ref[0] = x_ref[0] + lim
        o_ref[0] = off

    return pl.pallas_call(
        body,
        grid=(bsz,),
        in_specs=[pl.BlockSpec((1, s, 3), lambda b: (b, 0, 0)),
                  pl.BlockSpec((1, s, c), lambda b: (b, 0, 0)),
                  pl.BlockSpec(w1.shape, lambda b: (0, 0)),
                  pl.BlockSpec(b1r.shape, lambda b: (0, 0)),
                  pl.BlockSpec(wr3.shape, lambda b: (0, 0)),
                  pl.BlockSpec(br3.shape, lambda b: (0, 0))],
        out_specs=[pl.BlockSpec((1, s, 3), lambda b: (b, 0, 0)),
                   pl.BlockSpec((1, s, 3), lambda b: (b, 0, 0))],
        out_shape=[jax.ShapeDtypeStruct((bsz, s, 3), jnp.float32),
                   jax.ShapeDtypeStruct((bsz, s, 3), jnp.float32)],
    )(xyz, feats, w1, b1r, wr3, br3)


def _sa_layer(xyz, feats, npoint, li, p, ctr_xyz, dpad):
    cin = feats.shape[-1]
    if ctr_xyz is None:
        new_xyz = _fps(xyz, npoint)
    else:
        new_xyz = ctr_xyz
    gi, dv = _knn(new_xyz, xyz, np.float32(_RADII[li][1] ** 2))
    gath = _gather(xyz, feats, gi, dpad)
    nf = _sa_mlp(gath, new_xyz, dv, p, cin, _RADII[li])
    return new_xyz, nf


def kernel(points, params):
    xyz0 = points[:, 1:4].reshape(_B, _N, 3)
    feat0 = points[:, 4:].reshape(_B, _N, -1)
    sa = params['sa']

    nx1, f1 = _sa_layer(xyz0, feat0, 2048, 0, sa[0], None, 128)
    nx2, f2 = _sa_layer(nx1, f1, 512, 1, sa[1], None, 128)
    nx3, f3 = _sa_layer(nx2, f2, 256, 2, sa[2], None, 256)
    centers, offs = _vote(nx3, f3, params['vote'])
    _, cf = _sa_layer(nx3, f3, None, 3, sa[3], centers, 384)

    s = centers.shape[1]
    bidx = jnp.repeat(jnp.arange(_B, dtype=jnp.float32), s)[:, None]
    centers_cat = jnp.concatenate([bidx, centers.reshape(-1, 3)], axis=1)
    origin_cat = jnp.concatenate([bidx, nx3.reshape(-1, 3)], axis=1)
    off_cat = jnp.concatenate([bidx, offs.reshape(-1, 3)], axis=1)
    cf2 = cf.reshape(_B * s, -1)
    return centers_cat, origin_cat, off_cat, cf2
